# Initial kernel scaffold; baseline (speedup 1.0000x reference)
import jax
import jax.numpy as jnp
from jax.experimental import pallas as pl


def kernel(points, params):
    raise NotImplementedError("write your pallas kernel here")


# speedup vs baseline: 13.9471x; 13.9471x over previous
import functools

import jax
import jax.numpy as jnp
from jax import lax
from jax.experimental import pallas as pl
from jax.experimental.pallas import tpu as pltpu
from jax.experimental.pallas import tpu_sc as plsc

B, N, D, K, DEPTH = 4, 4096, 128, 16, 4
HID, NCLS = 256, 13
BN = B * N
DC = 136
EPS = 1e-5


def _dot(a, b):
    return jnp.dot(a, b, preferred_element_type=jnp.float32)


def _lnk(x, g, b):
    m = jnp.mean(x, axis=-1, keepdims=True)
    v = jnp.mean((x - m) * (x - m), axis=-1, keepdims=True)
    return (x - m) / jnp.sqrt(v + EPS) * g + b


_QB = 256


def _knn_body(q_ref, cT_ref, out_ref):
    b = pl.program_id(0)
    q = q_ref[0]
    c = cT_ref[0]
    qsq = (q[:, 0:1] * q[:, 0:1] + q[:, 2:3] * q[:, 2:3]) + q[:, 1:2] * q[:, 1:2]
    csq = (c[0:1] * c[0:1] + c[2:3] * c[2:3]) + c[1:2] * c[1:2]
    d2 = qsq + csq - 2.0 * jnp.dot(q, c, preferred_element_type=jnp.float32)
    colidx = lax.broadcasted_iota(jnp.int32, d2.shape, 1)
    kcol = lax.broadcasted_iota(jnp.int32, (_QB, K), 1)
    acc = jnp.zeros((_QB, K), jnp.int32)
    big = jnp.int32(2**30)
    inf = jnp.float32(jnp.inf)
    for k in range(K):
        m = jnp.min(d2, axis=1, keepdims=True)
        am = jnp.min(jnp.where(d2 == m, colidx, big), axis=1, keepdims=True)
        acc = jnp.where(kcol == k, am, acc)
        d2 = jnp.where(colidx == am, inf, d2)
    out_ref[0] = acc + b * N


def _knn(pts, ptsT):
    return pl.pallas_call(
        _knn_body,
        grid=(B, N // _QB),
        in_specs=[
            pl.BlockSpec((1, _QB, 3), lambda b, q: (b, q, 0)),
            pl.BlockSpec((1, 3, N), lambda b, q: (b, 0, 0)),
        ],
        out_specs=pl.BlockSpec((1, _QB, K), lambda b, q: (b, q, 0)),
        out_shape=jax.ShapeDtypeStruct((B, N, K), jnp.int32),
    )(pts, ptsT)


def _stem_body(p_ref, w1_ref, g1_ref, b1_ref, w2_ref, g2_ref, b2_ref, x_ref):
    p = p_ref[...]
    x = jax.nn.relu(_lnk(_dot(p, w1_ref[...]), g1_ref[...], b1_ref[...]))
    x_ref[...] = jax.nn.relu(_lnk(_dot(x, w2_ref[...]), g2_ref[...], b2_ref[...]))


def _stem(pts2, w1, g1, b1, w2, g2, b2):
    rb = 2048
    full = lambda a: pl.BlockSpec(a.shape, lambda r: tuple(0 for _ in a.shape))
    return pl.pallas_call(
        _stem_body,
        grid=(BN // rb,),
        in_specs=[pl.BlockSpec((rb, 3), lambda r: (r, 0))]
        + [full(a) for a in (w1, g1, b1, w2, g2, b2)],
        out_specs=pl.BlockSpec((rb, D), lambda r: (r, 0)),
        out_shape=jax.ShapeDtypeStruct((BN, D), jnp.float32),
    )(pts2, w1, g1, b1, w2, g2, b2)


_NW = 32
_PW = BN // _NW
_CPTS = 8
_NCH = _PW // _CPTS


def _sc_gather_body(x_hbm, gidx_hbm, dd_hbm, idx_v, rows_v, sem):
    c = lax.axis_index("c")
    s = lax.axis_index("s")
    w = s * 2 + c
    base_pt = w * _PW

    def chunk_body(ch, carry):
        pbase = base_pt + ch * _CPTS
        pltpu.sync_copy(gidx_hbm.at[pl.ds(pbase * K, _CPTS * K)], idx_v)
        pltpu.async_copy(x_hbm.at[idx_v], rows_v, sem).wait()
        pltpu.sync_copy(rows_v, dd_hbm.at[pl.ds(pbase * K, _CPTS * K)])
        return carry

    lax.fori_loop(0, _NCH, chunk_body, 0)


def _sc_gather(x, gidx_flat):
    return pl.kernel(
        _sc_gather_body,
        out_type=jax.ShapeDtypeStruct((BN * K, D), jnp.float32),
        mesh=plsc.VectorSubcoreMesh(core_axis_name="c", subcore_axis_name="s"),
        compiler_params=pltpu.CompilerParams(needs_layout_passes=False),
        scratch_types=[
            pltpu.VMEM((_CPTS * K,), jnp.int32),
            pltpu.VMEM((_CPTS * K, D), jnp.float32),
            pltpu.SemaphoreType.DMA,
        ],
    )(x, gidx_flat)


def _sc_rel_body(px_hbm, py_hbm, pz_hbm, gidx_hbm, rel_hbm,
                 px_v, py_v, pz_v, idx_v, rel_v, sem):
    c = lax.axis_index("c")
    s = lax.axis_index("s")
    w = s * 2 + c
    base_pt = w * _PW
    pltpu.sync_copy(px_hbm, px_v)
    pltpu.sync_copy(py_hbm, py_v)
    pltpu.sync_copy(pz_hbm, pz_v)
    zeros = jnp.zeros((16,), jnp.float32)
    for g in range(_CPTS):
        for j in range(D // 16):
            rel_v[g, pl.ds(16 * j, 16)] = zeros
    lanes = lax.iota(jnp.int32, 16) * 8

    def chunk_body(ch, carry):
        pbase = base_pt + ch * _CPTS
        pltpu.sync_copy(gidx_hbm.at[pl.ds(pbase * K, _CPTS * K)], idx_v)
        for g in range(_CPTS):
            gv = jnp.full((16,), g, jnp.int32)
            iv = idx_v[pl.ds(16 * g, 16)]
            cidx = jnp.full((16,), pbase + g, jnp.int32)
            for comp, table in ((0, px_v), (1, py_v), (2, pz_v)):
                nb = plsc.load_gather(table, [iv])
                cn = plsc.load_gather(table, [cidx])
                plsc.store_scatter(rel_v, [gv, lanes + comp], nb - cn)
        pltpu.sync_copy(rel_v, rel_hbm.at[pl.ds(pbase, _CPTS)])
        return carry

    lax.fori_loop(0, _NCH, chunk_body, 0)


def _sc_rel(px, py, pz, gidx_flat):
    return pl.kernel(
        _sc_rel_body,
        out_type=jax.ShapeDtypeStruct((BN, D), jnp.float32),
        mesh=plsc.VectorSubcoreMesh(core_axis_name="c", subcore_axis_name="s"),
        compiler_params=pltpu.CompilerParams(needs_layout_passes=False),
        scratch_types=[
            pltpu.VMEM((BN,), jnp.float32),
            pltpu.VMEM((BN,), jnp.float32),
            pltpu.VMEM((BN,), jnp.float32),
            pltpu.VMEM((_CPTS * K,), jnp.int32),
            pltpu.VMEM((_CPTS, D), jnp.float32),
            pltpu.SemaphoreType.DMA,
        ],
    )(px, py, pz, gidx_flat)


_RB = 512


def _blk_core(dd_ref, x_ref, rel_ref, wm_ref, lg_ref, lb_ref,
              g1_ref, b1_ref, w1_ref, w2_ref, g2_ref, b2_ref):
    x_in = x_ref[...]
    d = dd_ref[...].reshape(_RB, K, D) - x_in[:, None, :]
    rel8 = rel_ref[...].reshape(_RB, K, 8)
    cat = jnp.concatenate([d, rel8], axis=2).reshape(_RB * K, DC)
    msg = _dot(cat, wm_ref[...])
    msg = jax.nn.relu(_lnk(msg, lg_ref[...], lb_ref[...]))
    agg = jnp.max(msg.reshape(_RB, K, D), axis=1)
    x = _lnk(x_in + agg, g1_ref[...], b1_ref[...])
    h = _dot(jax.nn.relu(_dot(x, w1_ref[...])), w2_ref[...])
    return _lnk(x + h, g2_ref[...], b2_ref[...])


def _blk_mid_body(dd_ref, x_ref, rel_ref, wm_ref, lg_ref, lb_ref,
                  g1_ref, b1_ref, w1_ref, w2_ref, g2_ref, b2_ref, xo_ref):
    xo_ref[...] = _blk_core(dd_ref, x_ref, rel_ref, wm_ref, lg_ref, lb_ref,
                            g1_ref, b1_ref, w1_ref, w2_ref, g2_ref, b2_ref)


def _blk_last_body(dd_ref, x_ref, rel_ref, wm_ref, lg_ref, lb_ref,
                   g1_ref, b1_ref, w1_ref, w2_ref, g2_ref, b2_ref,
                   ng_ref, nb_ref, xo_ref):
    x = _blk_core(dd_ref, x_ref, rel_ref, wm_ref, lg_ref, lb_ref,
                  g1_ref, b1_ref, w1_ref, w2_ref, g2_ref, b2_ref)
    xo_ref[...] = _lnk(x, ng_ref[...], nb_ref[...])


def _blk(dd, x, rel, wm, lg, lb, g1, b1, w1, w2, g2, b2, norm=None):
    full = lambda a: pl.BlockSpec(a.shape, lambda r: tuple(0 for _ in a.shape))
    ins = [dd, x, rel, wm, lg, lb, g1, b1, w1, w2, g2, b2]
    body = _blk_mid_body
    if norm is not None:
        ins += list(norm)
        body = _blk_last_body
    return pl.pallas_call(
        body,
        grid=(BN // _RB,),
        in_specs=[pl.BlockSpec((_RB * K, D), lambda r: (r, 0)),
                  pl.BlockSpec((_RB, D), lambda r: (r, 0)),
                  pl.BlockSpec((_RB, D), lambda r: (r, 0))]
        + [full(a) for a in ins[3:]],
        out_specs=pl.BlockSpec((_RB, D), lambda r: (r, 0)),
        out_shape=jax.ShapeDtypeStruct((BN, D), jnp.float32),
    )(*ins)


def _gmax_body(x_ref, o_ref):
    o_ref[...] = jnp.max(x_ref[...].reshape(B, N, D), axis=1)


def _gmax(xfin):
    return pl.pallas_call(
        _gmax_body,
        in_specs=[pl.BlockSpec((BN, D), lambda: (0, 0))],
        out_specs=pl.BlockSpec((B, D), lambda: (0, 0)),
        out_shape=jax.ShapeDtypeStruct((B, D), jnp.float32),
    )(xfin)


def _head_body(e_ref, x_ref, g_ref, wa_ref, wb_ref, wc_ref, bg1_ref, bb1_ref,
               w2_ref, bg2_ref, bb2_ref, w3_ref, b3_ref, o_ref):
    gb = g_ref[pl.ds(pl.program_id(0), 1), :]
    h = (_dot(e_ref[...], wa_ref[...]) + _dot(x_ref[...], wb_ref[...])
         + _dot(gb, wc_ref[...]))
    h = jax.nn.relu(h * bg1_ref[...] + bb1_ref[...])
    h = _dot(h, w2_ref[...])
    h = jax.nn.relu(h * bg2_ref[...] + bb2_ref[...])
    o_ref[...] = _dot(h, w3_ref[...]) + b3_ref[...]


def _head(early, xfin, g, wa, wb, wc, bg1, bb1, w2, bg2, bb2, w3, b3):
    rb = 2048
    row = pl.BlockSpec((rb, D), lambda b, r: (b * (N // rb) + r, 0))
    full = lambda a: pl.BlockSpec(a.shape, lambda b, r: tuple(0 for _ in a.shape))
    return pl.pallas_call(
        _head_body,
        grid=(B, N // rb),
        in_specs=[row, row, pl.BlockSpec((B, D), lambda b, r: (0, 0))]
        + [full(a) for a in (wa, wb, wc, bg1, bb1, w2, bg2, bb2, w3, b3)],
        out_specs=pl.BlockSpec((rb, NCLS), lambda b, r: (b * (N // rb) + r, 0)),
        out_shape=jax.ShapeDtypeStruct((BN, NCLS), jnp.float32),
    )(early, xfin, g, wa, wb, wc, bg1, bb1, w2, bg2, bb2, w3, b3)


def kernel(points, params):
    pts = points
    pts2 = pts.reshape(BN, 3)
    ptsT = pts.transpose(0, 2, 1)
    blocks = params['blocks']
    r2 = lambda a: a.reshape(1, -1)
    pad_wm = lambda w: jnp.concatenate(
        [w, jnp.zeros((DC - w.shape[0], D), jnp.float32)], axis=0)

    gidx = _knn(pts, ptsT)
    gidx_flat = gidx.reshape(-1)
    rel = _sc_rel(pts2[:, 0], pts2[:, 1], pts2[:, 2], gidx_flat)

    x = _stem(pts2,
              params['stem_w1'], r2(params['stem_ln1_g']), r2(params['stem_ln1_b']),
              params['stem_w2'], r2(params['stem_ln2_g']), r2(params['stem_ln2_b']))

    early = None
    for i in range(DEPTH):
        blk = blocks[i]
        dd = _sc_gather(x, gidx_flat)
        norm = (r2(params['norm_g']), r2(params['norm_b'])) if i == DEPTH - 1 else None
        x = _blk(dd, x, rel,
                 pad_wm(blk['wm']), r2(blk['lnm_g']), r2(blk['lnm_b']),
                 r2(blk['ln1_g']), r2(blk['ln1_b']),
                 blk['ffn_w1'], blk['ffn_w2'],
                 r2(blk['ln2_g']), r2(blk['ln2_b']), norm=norm)
        if i == DEPTH // 2 - 1:
            early = x

    xfin = x
    g = _gmax(xfin)
    h1 = params['h1_w']
    out = _head(early, xfin, g,
                h1[:D], h1[D:2 * D], h1[2 * D:],
                r2(params['h1_bn_g']), r2(params['h1_bn_b']),
                params['h2_w'],
                r2(params['h2_bn_g']), r2(params['h2_bn_b']),
                params['h3_w'], r2(params['h3_b']))
    return out.reshape(B, N, NCLS).transpose(0, 2, 1)


# speedup vs baseline: 15.6358x; 1.1211x over previous
import functools

import jax
import jax.numpy as jnp
from jax import lax
from jax.experimental import pallas as pl
from jax.experimental.pallas import tpu as pltpu
from jax.experimental.pallas import tpu_sc as plsc

B, N, D, K, DEPTH = 4, 4096, 128, 16, 4
HID, NCLS = 256, 13
BN = B * N
DC = 136
EPS = 1e-5


def _dot(a, b):
    return jnp.dot(a, b, preferred_element_type=jnp.float32)


def _lnk(x, g, b):
    m = jnp.mean(x, axis=-1, keepdims=True)
    v = jnp.mean((x - m) * (x - m), axis=-1, keepdims=True)
    return (x - m) / jnp.sqrt(v + EPS) * g + b


_QB = 256


def _knn_body(q_ref, cT_ref, out_ref):
    b = pl.program_id(0)
    q = q_ref[0]
    c = cT_ref[0]
    qsq = (q[:, 0:1] * q[:, 0:1] + q[:, 2:3] * q[:, 2:3]) + q[:, 1:2] * q[:, 1:2]
    csq = (c[0:1] * c[0:1] + c[2:3] * c[2:3]) + c[1:2] * c[1:2]
    d2 = qsq + csq - 2.0 * jnp.dot(q, c, preferred_element_type=jnp.float32)
    colidx = lax.broadcasted_iota(jnp.int32, d2.shape, 1)
    kcol = lax.broadcasted_iota(jnp.int32, (_QB, K), 1)
    acc = jnp.zeros((_QB, K), jnp.int32)
    big = jnp.int32(2**30)
    inf = jnp.float32(jnp.inf)
    for k in range(K):
        m = jnp.min(d2, axis=1, keepdims=True)
        am = jnp.min(jnp.where(d2 == m, colidx, big), axis=1, keepdims=True)
        acc = jnp.where(kcol == k, am, acc)
        d2 = jnp.where(colidx == am, inf, d2)
    out_ref[0] = acc + b * N


def _knn(pts, ptsT):
    return pl.pallas_call(
        _knn_body,
        grid=(B, N // _QB),
        in_specs=[
            pl.BlockSpec((1, _QB, 3), lambda b, q: (b, q, 0)),
            pl.BlockSpec((1, 3, N), lambda b, q: (b, 0, 0)),
        ],
        out_specs=pl.BlockSpec((1, _QB, K), lambda b, q: (b, q, 0)),
        out_shape=jax.ShapeDtypeStruct((B, N, K), jnp.int32),
    )(pts, ptsT)


def _stem_body(p_ref, w1_ref, g1_ref, b1_ref, w2_ref, g2_ref, b2_ref, x_ref):
    p = p_ref[...]
    x = jax.nn.relu(_lnk(_dot(p, w1_ref[...]), g1_ref[...], b1_ref[...]))
    x_ref[...] = jax.nn.relu(_lnk(_dot(x, w2_ref[...]), g2_ref[...], b2_ref[...]))


def _stem(pts2, w1, g1, b1, w2, g2, b2):
    rb = 2048
    full = lambda a: pl.BlockSpec(a.shape, lambda r: tuple(0 for _ in a.shape))
    return pl.pallas_call(
        _stem_body,
        grid=(BN // rb,),
        in_specs=[pl.BlockSpec((rb, 3), lambda r: (r, 0))]
        + [full(a) for a in (w1, g1, b1, w2, g2, b2)],
        out_specs=pl.BlockSpec((rb, D), lambda r: (r, 0)),
        out_shape=jax.ShapeDtypeStruct((BN, D), jnp.float32),
    )(pts2, w1, g1, b1, w2, g2, b2)


_NW = 32
_PW = BN // _NW
_CPTS = 8
_NCH = _PW // _CPTS


def _sc_gather_body(x_hbm, gidx2_hbm, dd_hbm, idx_v, rows0_v, rows1_v,
                    sem0, sem1):
    c = lax.axis_index("c")
    s = lax.axis_index("s")
    w = s * 2 + c
    base_pt = w * _PW
    rows = (rows0_v, rows1_v)
    sems = (sem0, sem1)
    pltpu.sync_copy(gidx2_hbm.at[pl.ds(w * _NCH, _NCH)], idx_v)

    def fire(ch, buf):
        pltpu.async_copy(x_hbm.at[idx_v.at[ch]], rows[buf], sems[buf])

    fire(0, 0)
    fire(1, 1)

    def pair_body(t, carry):
        ch = 2 * t
        for buf in range(2):
            chb = ch + buf
            pltpu.make_async_copy(x_hbm.at[idx_v.at[chb]], rows[buf],
                                  sems[buf]).wait()
            pltpu.sync_copy(
                rows[buf],
                dd_hbm.at[pl.ds((base_pt + chb * _CPTS) * K, _CPTS * K)])

            @pl.when(chb + 2 < _NCH)
            def _():
                fire(chb + 2, buf)
        return carry

    lax.fori_loop(0, _NCH // 2, pair_body, 0)


def _sc_gather(x, gidx2):
    return pl.kernel(
        _sc_gather_body,
        out_type=jax.ShapeDtypeStruct((BN * K, D), jnp.float32),
        mesh=plsc.VectorSubcoreMesh(core_axis_name="c", subcore_axis_name="s"),
        compiler_params=pltpu.CompilerParams(needs_layout_passes=False),
        scratch_types=[
            pltpu.VMEM((_NCH, _CPTS * K), jnp.int32),
            pltpu.VMEM((_CPTS * K, D), jnp.float32),
            pltpu.VMEM((_CPTS * K, D), jnp.float32),
            pltpu.SemaphoreType.DMA,
            pltpu.SemaphoreType.DMA,
        ],
    )(x, gidx2)


def _sc_rel_body(px_hbm, py_hbm, pz_hbm, gidx_hbm, rel_hbm,
                 px_v, py_v, pz_v, idx_v, rel_v, sem):
    c = lax.axis_index("c")
    s = lax.axis_index("s")
    w = s * 2 + c
    base_pt = w * _PW
    pltpu.sync_copy(px_hbm, px_v)
    pltpu.sync_copy(py_hbm, py_v)
    pltpu.sync_copy(pz_hbm, pz_v)
    zeros = jnp.zeros((16,), jnp.float32)
    for g in range(_CPTS):
        for j in range(D // 16):
            rel_v[g, pl.ds(16 * j, 16)] = zeros
    lanes = lax.iota(jnp.int32, 16) * 8

    def chunk_body(ch, carry):
        pbase = base_pt + ch * _CPTS
        pltpu.sync_copy(gidx_hbm.at[pl.ds(pbase * K, _CPTS * K)], idx_v)
        for g in range(_CPTS):
            gv = jnp.full((16,), g, jnp.int32)
            iv = idx_v[pl.ds(16 * g, 16)]
            cidx = jnp.full((16,), pbase + g, jnp.int32)
            for comp, table in ((0, px_v), (1, py_v), (2, pz_v)):
                nb = plsc.load_gather(table, [iv])
                cn = plsc.load_gather(table, [cidx])
                plsc.store_scatter(rel_v, [gv, lanes + comp], nb - cn)
        pltpu.sync_copy(rel_v, rel_hbm.at[pl.ds(pbase, _CPTS)])
        return carry

    lax.fori_loop(0, _NCH, chunk_body, 0)


def _sc_rel(px, py, pz, gidx_flat):
    return pl.kernel(
        _sc_rel_body,
        out_type=jax.ShapeDtypeStruct((BN, D), jnp.float32),
        mesh=plsc.VectorSubcoreMesh(core_axis_name="c", subcore_axis_name="s"),
        compiler_params=pltpu.CompilerParams(needs_layout_passes=False),
        scratch_types=[
            pltpu.VMEM((BN,), jnp.float32),
            pltpu.VMEM((BN,), jnp.float32),
            pltpu.VMEM((BN,), jnp.float32),
            pltpu.VMEM((_CPTS * K,), jnp.int32),
            pltpu.VMEM((_CPTS, D), jnp.float32),
            pltpu.SemaphoreType.DMA,
        ],
    )(px, py, pz, gidx_flat)


_RB = 512


def _blk_core(dd_ref, x_ref, rel_ref, wm_ref, lg_ref, lb_ref,
              g1_ref, b1_ref, w1_ref, w2_ref, g2_ref, b2_ref):
    x_in = x_ref[...]
    d = dd_ref[...].reshape(_RB, K, D) - x_in[:, None, :]
    rel8 = rel_ref[...].reshape(_RB, K, 8)
    cat = jnp.concatenate([d, rel8], axis=2).reshape(_RB * K, DC)
    msg = _dot(cat, wm_ref[...])
    msg = jax.nn.relu(_lnk(msg, lg_ref[...], lb_ref[...]))
    agg = jnp.max(msg.reshape(_RB, K, D), axis=1)
    x = _lnk(x_in + agg, g1_ref[...], b1_ref[...])
    h = _dot(jax.nn.relu(_dot(x, w1_ref[...])), w2_ref[...])
    return _lnk(x + h, g2_ref[...], b2_ref[...])


def _blk_mid_body(dd_ref, x_ref, rel_ref, wm_ref, lg_ref, lb_ref,
                  g1_ref, b1_ref, w1_ref, w2_ref, g2_ref, b2_ref, xo_ref):
    xo_ref[...] = _blk_core(dd_ref, x_ref, rel_ref, wm_ref, lg_ref, lb_ref,
                            g1_ref, b1_ref, w1_ref, w2_ref, g2_ref, b2_ref)


def _blk_last_body(dd_ref, x_ref, rel_ref, wm_ref, lg_ref, lb_ref,
                   g1_ref, b1_ref, w1_ref, w2_ref, g2_ref, b2_ref,
                   ng_ref, nb_ref, xo_ref):
    x = _blk_core(dd_ref, x_ref, rel_ref, wm_ref, lg_ref, lb_ref,
                  g1_ref, b1_ref, w1_ref, w2_ref, g2_ref, b2_ref)
    xo_ref[...] = _lnk(x, ng_ref[...], nb_ref[...])


def _blk(dd, x, rel, wm, lg, lb, g1, b1, w1, w2, g2, b2, norm=None):
    full = lambda a: pl.BlockSpec(a.shape, lambda r: tuple(0 for _ in a.shape))
    ins = [dd, x, rel, wm, lg, lb, g1, b1, w1, w2, g2, b2]
    body = _blk_mid_body
    if norm is not None:
        ins += list(norm)
        body = _blk_last_body
    return pl.pallas_call(
        body,
        grid=(BN // _RB,),
        in_specs=[pl.BlockSpec((_RB * K, D), lambda r: (r, 0)),
                  pl.BlockSpec((_RB, D), lambda r: (r, 0)),
                  pl.BlockSpec((_RB, D), lambda r: (r, 0))]
        + [full(a) for a in ins[3:]],
        out_specs=pl.BlockSpec((_RB, D), lambda r: (r, 0)),
        out_shape=jax.ShapeDtypeStruct((BN, D), jnp.float32),
    )(*ins)


def _gmax_body(x_ref, o_ref):
    o_ref[...] = jnp.max(x_ref[...].reshape(B, N, D), axis=1)


def _gmax(xfin):
    return pl.pallas_call(
        _gmax_body,
        in_specs=[pl.BlockSpec((BN, D), lambda: (0, 0))],
        out_specs=pl.BlockSpec((B, D), lambda: (0, 0)),
        out_shape=jax.ShapeDtypeStruct((B, D), jnp.float32),
    )(xfin)


def _head_body(e_ref, x_ref, g_ref, wa_ref, wb_ref, wc_ref, bg1_ref, bb1_ref,
               w2_ref, bg2_ref, bb2_ref, w3_ref, b3_ref, o_ref):
    gb = g_ref[pl.ds(pl.program_id(0), 1), :]
    h = (_dot(e_ref[...], wa_ref[...]) + _dot(x_ref[...], wb_ref[...])
         + _dot(gb, wc_ref[...]))
    h = jax.nn.relu(h * bg1_ref[...] + bb1_ref[...])
    h = _dot(h, w2_ref[...])
    h = jax.nn.relu(h * bg2_ref[...] + bb2_ref[...])
    o_ref[...] = _dot(h, w3_ref[...]) + b3_ref[...]


def _head(early, xfin, g, wa, wb, wc, bg1, bb1, w2, bg2, bb2, w3, b3):
    rb = 2048
    row = pl.BlockSpec((rb, D), lambda b, r: (b * (N // rb) + r, 0))
    full = lambda a: pl.BlockSpec(a.shape, lambda b, r: tuple(0 for _ in a.shape))
    return pl.pallas_call(
        _head_body,
        grid=(B, N // rb),
        in_specs=[row, row, pl.BlockSpec((B, D), lambda b, r: (0, 0))]
        + [full(a) for a in (wa, wb, wc, bg1, bb1, w2, bg2, bb2, w3, b3)],
        out_specs=pl.BlockSpec((rb, NCLS), lambda b, r: (b * (N // rb) + r, 0)),
        out_shape=jax.ShapeDtypeStruct((BN, NCLS), jnp.float32),
    )(early, xfin, g, wa, wb, wc, bg1, bb1, w2, bg2, bb2, w3, b3)


def kernel(points, params):
    pts = points
    pts2 = pts.reshape(BN, 3)
    ptsT = pts.transpose(0, 2, 1)
    blocks = params['blocks']
    r2 = lambda a: a.reshape(1, -1)
    pad_wm = lambda w: jnp.concatenate(
        [w, jnp.zeros((DC - w.shape[0], D), jnp.float32)], axis=0)

    gidx = _knn(pts, ptsT)
    gidx_flat = gidx.reshape(-1)
    gidx2 = gidx_flat.reshape(-1, _CPTS * K)
    rel = _sc_rel(pts2[:, 0], pts2[:, 1], pts2[:, 2], gidx_flat)

    x = _stem(pts2,
              params['stem_w1'], r2(params['stem_ln1_g']), r2(params['stem_ln1_b']),
              params['stem_w2'], r2(params['stem_ln2_g']), r2(params['stem_ln2_b']))

    early = None
    for i in range(DEPTH):
        blk = blocks[i]
        dd = _sc_gather(x, gidx2)
        norm = (r2(params['norm_g']), r2(params['norm_b'])) if i == DEPTH - 1 else None
        x = _blk(dd, x, rel,
                 pad_wm(blk['wm']), r2(blk['lnm_g']), r2(blk['lnm_b']),
                 r2(blk['ln1_g']), r2(blk['ln1_b']),
                 blk['ffn_w1'], blk['ffn_w2'],
                 r2(blk['ln2_g']), r2(blk['ln2_b']), norm=norm)
        if i == DEPTH // 2 - 1:
            early = x

    xfin = x
    g = _gmax(xfin)
    h1 = params['h1_w']
    out = _head(early, xfin, g,
                h1[:D], h1[D:2 * D], h1[2 * D:],
                r2(params['h1_bn_g']), r2(params['h1_bn_b']),
                params['h2_w'],
                r2(params['h2_bn_g']), r2(params['h2_bn_b']),
                params['h3_w'], r2(params['h3_b']))
    return out.reshape(B, N, NCLS).transpose(0, 2, 1)


# speedup vs baseline: 15.9479x; 1.0200x over previous
import functools

import jax
import jax.numpy as jnp
from jax import lax
from jax.experimental import pallas as pl
from jax.experimental.pallas import tpu as pltpu
from jax.experimental.pallas import tpu_sc as plsc

B, N, D, K, DEPTH = 4, 4096, 128, 16, 4
HID, NCLS = 256, 13
BN = B * N
DC = 136
EPS = 1e-5


def _dot(a, b):
    return jnp.dot(a, b, preferred_element_type=jnp.float32)


def _lnk(x, g, b):
    m = jnp.mean(x, axis=-1, keepdims=True)
    v = jnp.mean((x - m) * (x - m), axis=-1, keepdims=True)
    return (x - m) / jnp.sqrt(v + EPS) * g + b


_QB = 512


def _knn_body(q_ref, cT_ref, out_ref):
    b = pl.program_id(0)
    q = q_ref[0]
    c = cT_ref[0]
    qsq = (q[:, 0:1] * q[:, 0:1] + q[:, 2:3] * q[:, 2:3]) + q[:, 1:2] * q[:, 1:2]
    csq = (c[0:1] * c[0:1] + c[2:3] * c[2:3]) + c[1:2] * c[1:2]
    d2 = qsq + csq - 2.0 * jnp.dot(q, c, preferred_element_type=jnp.float32)
    colidx = lax.broadcasted_iota(jnp.int32, d2.shape, 1)
    kcol = lax.broadcasted_iota(jnp.int32, (_QB, K), 1)
    acc = jnp.zeros((_QB, K), jnp.int32)
    big = jnp.int32(2**30)
    inf = jnp.float32(jnp.inf)
    for k in range(K):
        m = jnp.min(d2, axis=1, keepdims=True)
        am = jnp.min(jnp.where(d2 == m, colidx, big), axis=1, keepdims=True)
        acc = jnp.where(kcol == k, am, acc)
        d2 = jnp.where(colidx == am, inf, d2)
    out_ref[0] = acc + b * N


def _knn(pts, ptsT):
    return pl.pallas_call(
        _knn_body,
        grid=(B, N // _QB),
        in_specs=[
            pl.BlockSpec((1, _QB, 3), lambda b, q: (b, q, 0)),
            pl.BlockSpec((1, 3, N), lambda b, q: (b, 0, 0)),
        ],
        out_specs=pl.BlockSpec((1, _QB, K), lambda b, q: (b, q, 0)),
        out_shape=jax.ShapeDtypeStruct((B, N, K), jnp.int32),
    )(pts, ptsT)


def _stem_body(p_ref, w1_ref, g1_ref, b1_ref, w2_ref, g2_ref, b2_ref, x_ref):
    p = p_ref[...]
    x = jax.nn.relu(_lnk(_dot(p, w1_ref[...]), g1_ref[...], b1_ref[...]))
    x_ref[...] = jax.nn.relu(_lnk(_dot(x, w2_ref[...]), g2_ref[...], b2_ref[...]))


def _stem(pts2, w1, g1, b1, w2, g2, b2):
    rb = 2048
    full = lambda a: pl.BlockSpec(a.shape, lambda r: tuple(0 for _ in a.shape))
    return pl.pallas_call(
        _stem_body,
        grid=(BN // rb,),
        in_specs=[pl.BlockSpec((rb, 3), lambda r: (r, 0))]
        + [full(a) for a in (w1, g1, b1, w2, g2, b2)],
        out_specs=pl.BlockSpec((rb, D), lambda r: (r, 0)),
        out_shape=jax.ShapeDtypeStruct((BN, D), jnp.float32),
    )(pts2, w1, g1, b1, w2, g2, b2)


_NW = 32
_PW = BN // _NW
_CPTS = 8
_NCH = _PW // _CPTS


def _sc_gather_body(x_hbm, gidx2_hbm, dd_hbm, idx_v, rows0_v, rows1_v,
                    sem0, sem1):
    c = lax.axis_index("c")
    s = lax.axis_index("s")
    w = s * 2 + c
    base_pt = w * _PW
    rows = (rows0_v, rows1_v)
    sems = (sem0, sem1)
    pltpu.sync_copy(gidx2_hbm.at[pl.ds(w * _NCH, _NCH)], idx_v)

    def fire(ch, buf):
        pltpu.async_copy(x_hbm.at[idx_v.at[ch]], rows[buf], sems[buf])

    fire(0, 0)
    fire(1, 1)

    def pair_body(t, carry):
        ch = 2 * t
        for buf in range(2):
            chb = ch + buf
            pltpu.make_async_copy(x_hbm.at[idx_v.at[chb]], rows[buf],
                                  sems[buf]).wait()
            pltpu.sync_copy(
                rows[buf],
                dd_hbm.at[pl.ds((base_pt + chb * _CPTS) * K, _CPTS * K)])

            @pl.when(chb + 2 < _NCH)
            def _():
                fire(chb + 2, buf)
        return carry

    lax.fori_loop(0, _NCH // 2, pair_body, 0)


def _sc_gather(x, gidx2):
    return pl.kernel(
        _sc_gather_body,
        out_type=jax.ShapeDtypeStruct((BN * K, D), jnp.float32),
        mesh=plsc.VectorSubcoreMesh(core_axis_name="c", subcore_axis_name="s"),
        compiler_params=pltpu.CompilerParams(needs_layout_passes=False),
        scratch_types=[
            pltpu.VMEM((_NCH, _CPTS * K), jnp.int32),
            pltpu.VMEM((_CPTS * K, D), jnp.float32),
            pltpu.VMEM((_CPTS * K, D), jnp.float32),
            pltpu.SemaphoreType.DMA,
            pltpu.SemaphoreType.DMA,
        ],
    )(x, gidx2)


def _sc_rel_body(px_hbm, py_hbm, pz_hbm, gidx_hbm, rel_hbm,
                 px_v, py_v, pz_v, idx_v, rel_v, sem):
    c = lax.axis_index("c")
    s = lax.axis_index("s")
    w = s * 2 + c
    base_pt = w * _PW
    pltpu.sync_copy(px_hbm, px_v)
    pltpu.sync_copy(py_hbm, py_v)
    pltpu.sync_copy(pz_hbm, pz_v)
    zeros = jnp.zeros((16,), jnp.float32)
    for g in range(_CPTS):
        for j in range(D // 16):
            rel_v[g, pl.ds(16 * j, 16)] = zeros
    lanes = lax.iota(jnp.int32, 16) * 8

    def chunk_body(ch, carry):
        pbase = base_pt + ch * _CPTS
        pltpu.sync_copy(gidx_hbm.at[pl.ds(pbase * K, _CPTS * K)], idx_v)
        for g in range(_CPTS):
            gv = jnp.full((16,), g, jnp.int32)
            iv = idx_v[pl.ds(16 * g, 16)]
            cidx = jnp.full((16,), pbase + g, jnp.int32)
            for comp, table in ((0, px_v), (1, py_v), (2, pz_v)):
                nb = plsc.load_gather(table, [iv])
                cn = plsc.load_gather(table, [cidx])
                plsc.store_scatter(rel_v, [gv, lanes + comp], nb - cn)
        pltpu.sync_copy(rel_v, rel_hbm.at[pl.ds(pbase, _CPTS)])
        return carry

    lax.fori_loop(0, _NCH, chunk_body, 0)


def _sc_rel(px, py, pz, gidx_flat):
    return pl.kernel(
        _sc_rel_body,
        out_type=jax.ShapeDtypeStruct((BN, D), jnp.float32),
        mesh=plsc.VectorSubcoreMesh(core_axis_name="c", subcore_axis_name="s"),
        compiler_params=pltpu.CompilerParams(needs_layout_passes=False),
        scratch_types=[
            pltpu.VMEM((BN,), jnp.float32),
            pltpu.VMEM((BN,), jnp.float32),
            pltpu.VMEM((BN,), jnp.float32),
            pltpu.VMEM((_CPTS * K,), jnp.int32),
            pltpu.VMEM((_CPTS, D), jnp.float32),
            pltpu.SemaphoreType.DMA,
        ],
    )(px, py, pz, gidx_flat)


_RB = 512


def _blk_core(dd_ref, x_ref, rel_ref, wm_ref, lg_ref, lb_ref,
              g1_ref, b1_ref, w1_ref, w2_ref, g2_ref, b2_ref):
    x_in = x_ref[...]
    d = dd_ref[...].reshape(_RB, K, D) - x_in[:, None, :]
    rel8 = rel_ref[...].reshape(_RB, K, 8)
    cat = jnp.concatenate([d, rel8], axis=2).reshape(_RB * K, DC)
    msg = _dot(cat, wm_ref[...])
    msg = jax.nn.relu(_lnk(msg, lg_ref[...], lb_ref[...]))
    agg = jnp.max(msg.reshape(_RB, K, D), axis=1)
    x = _lnk(x_in + agg, g1_ref[...], b1_ref[...])
    h = _dot(jax.nn.relu(_dot(x, w1_ref[...])), w2_ref[...])
    return _lnk(x + h, g2_ref[...], b2_ref[...])


def _blk_mid_body(dd_ref, x_ref, rel_ref, wm_ref, lg_ref, lb_ref,
                  g1_ref, b1_ref, w1_ref, w2_ref, g2_ref, b2_ref, xo_ref):
    xo_ref[...] = _blk_core(dd_ref, x_ref, rel_ref, wm_ref, lg_ref, lb_ref,
                            g1_ref, b1_ref, w1_ref, w2_ref, g2_ref, b2_ref)


def _blk_last_body(dd_ref, x_ref, rel_ref, wm_ref, lg_ref, lb_ref,
                   g1_ref, b1_ref, w1_ref, w2_ref, g2_ref, b2_ref,
                   ng_ref, nb_ref, xo_ref):
    x = _blk_core(dd_ref, x_ref, rel_ref, wm_ref, lg_ref, lb_ref,
                  g1_ref, b1_ref, w1_ref, w2_ref, g2_ref, b2_ref)
    xo_ref[...] = _lnk(x, ng_ref[...], nb_ref[...])


def _blk(dd, x, rel, wm, lg, lb, g1, b1, w1, w2, g2, b2, norm=None):
    full = lambda a: pl.BlockSpec(a.shape, lambda r: tuple(0 for _ in a.shape))
    ins = [dd, x, rel, wm, lg, lb, g1, b1, w1, w2, g2, b2]
    body = _blk_mid_body
    if norm is not None:
        ins += list(norm)
        body = _blk_last_body
    return pl.pallas_call(
        body,
        grid=(BN // _RB,),
        in_specs=[pl.BlockSpec((_RB * K, D), lambda r: (r, 0)),
                  pl.BlockSpec((_RB, D), lambda r: (r, 0)),
                  pl.BlockSpec((_RB, D), lambda r: (r, 0))]
        + [full(a) for a in ins[3:]],
        out_specs=pl.BlockSpec((_RB, D), lambda r: (r, 0)),
        out_shape=jax.ShapeDtypeStruct((BN, D), jnp.float32),
    )(*ins)


def _gmax_body(x_ref, o_ref):
    o_ref[...] = jnp.max(x_ref[...].reshape(B, N, D), axis=1)


def _gmax(xfin):
    return pl.pallas_call(
        _gmax_body,
        in_specs=[pl.BlockSpec((BN, D), lambda: (0, 0))],
        out_specs=pl.BlockSpec((B, D), lambda: (0, 0)),
        out_shape=jax.ShapeDtypeStruct((B, D), jnp.float32),
    )(xfin)


def _head_body(e_ref, x_ref, g_ref, wa_ref, wb_ref, wc_ref, bg1_ref, bb1_ref,
               w2_ref, bg2_ref, bb2_ref, w3_ref, b3_ref, o_ref):
    gb = g_ref[pl.ds(pl.program_id(0), 1), :]
    h = (_dot(e_ref[...], wa_ref[...]) + _dot(x_ref[...], wb_ref[...])
         + _dot(gb, wc_ref[...]))
    h = jax.nn.relu(h * bg1_ref[...] + bb1_ref[...])
    h = _dot(h, w2_ref[...])
    h = jax.nn.relu(h * bg2_ref[...] + bb2_ref[...])
    o_ref[...] = _dot(h, w3_ref[...]) + b3_ref[...]


def _head(early, xfin, g, wa, wb, wc, bg1, bb1, w2, bg2, bb2, w3, b3):
    rb = 2048
    row = pl.BlockSpec((rb, D), lambda b, r: (b * (N // rb) + r, 0))
    full = lambda a: pl.BlockSpec(a.shape, lambda b, r: tuple(0 for _ in a.shape))
    return pl.pallas_call(
        _head_body,
        grid=(B, N // rb),
        in_specs=[row, row, pl.BlockSpec((B, D), lambda b, r: (0, 0))]
        + [full(a) for a in (wa, wb, wc, bg1, bb1, w2, bg2, bb2, w3, b3)],
        out_specs=pl.BlockSpec((rb, NCLS), lambda b, r: (b * (N // rb) + r, 0)),
        out_shape=jax.ShapeDtypeStruct((BN, NCLS), jnp.float32),
    )(early, xfin, g, wa, wb, wc, bg1, bb1, w2, bg2, bb2, w3, b3)


def kernel(points, params):
    pts = points
    pts2 = pts.reshape(BN, 3)
    ptsT = pts.transpose(0, 2, 1)
    blocks = params['blocks']
    r2 = lambda a: a.reshape(1, -1)
    pad_wm = lambda w: jnp.concatenate(
        [w, jnp.zeros((DC - w.shape[0], D), jnp.float32)], axis=0)

    gidx = _knn(pts, ptsT)
    gidx_flat = gidx.reshape(-1)
    gidx2 = gidx_flat.reshape(-1, _CPTS * K)
    rel = _sc_rel(pts2[:, 0], pts2[:, 1], pts2[:, 2], gidx_flat)

    x = _stem(pts2,
              params['stem_w1'], r2(params['stem_ln1_g']), r2(params['stem_ln1_b']),
              params['stem_w2'], r2(params['stem_ln2_g']), r2(params['stem_ln2_b']))

    early = None
    for i in range(DEPTH):
        blk = blocks[i]
        dd = _sc_gather(x, gidx2)
        norm = (r2(params['norm_g']), r2(params['norm_b'])) if i == DEPTH - 1 else None
        x = _blk(dd, x, rel,
                 pad_wm(blk['wm']), r2(blk['lnm_g']), r2(blk['lnm_b']),
                 r2(blk['ln1_g']), r2(blk['ln1_b']),
                 blk['ffn_w1'], blk['ffn_w2'],
                 r2(blk['ln2_g']), r2(blk['ln2_b']), norm=norm)
        if i == DEPTH // 2 - 1:
            early = x

    xfin = x
    g = _gmax(xfin)
    h1 = params['h1_w']
    out = _head(early, xfin, g,
                h1[:D], h1[D:2 * D], h1[2 * D:],
                r2(params['h1_bn_g']), r2(params['h1_bn_b']),
                params['h2_w'],
                r2(params['h2_bn_g']), r2(params['h2_bn_b']),
                params['h3_w'], r2(params['h3_b']))
    return out.reshape(B, N, NCLS).transpose(0, 2, 1)
